# Initial kernel scaffold; baseline (speedup 1.0000x reference)
import jax
import jax.numpy as jnp
from jax.experimental import pallas as pl


def kernel(opponent_policy, policy, wr, F, iter_num):
    raise NotImplementedError("write your pallas kernel here")


# speedup vs baseline: 415.3940x; 415.3940x over previous
import functools

import jax
import jax.numpy as jnp
from jax import lax
from jax.experimental import pallas as pl
from jax.experimental.pallas import tpu as pltpu
from jax.experimental.pallas import tpu_sc as plsc

MAX_HEALTH = 13
MAX_ENERGY = 13
ACT = 16
N_STATES = (MAX_HEALTH + 1) ** 2 * (MAX_ENERGY + 1) ** 2

NUM_WORKERS = 32
PER_W = 1216
N_PAD = NUM_WORKERS * PER_W
GROUPS = PER_W // 16
CHUNK_STATES = 64
CHUNK_GROUPS = CHUNK_STATES // 16
N_CHUNKS = PER_W // CHUNK_STATES

_MESH = plsc.VectorSubcoreMesh(core_axis_name="c", subcore_axis_name="s")


def _worker_id():
    return lax.axis_index("c") * 16 + lax.axis_index("s")


def _softmax_body(o_ref, p_ref, a_ref, b_ref):
    for src, dst in ((o_ref, a_ref), (p_ref, b_ref)):
        x = src[:]
        m = jnp.max(x, axis=1, keepdims=True)
        e = jnp.exp(x - m)
        dst[:] = e / jnp.sum(e, axis=1, keepdims=True)


def _softmax_prep(o_pad, p_pad):
    blk = pl.BlockSpec((512, ACT), lambda i: (i, 0))
    return pl.pallas_call(
        _softmax_body,
        grid=(N_PAD // 512,),
        in_specs=[blk, blk],
        out_specs=(blk, blk),
        out_shape=(
            jax.ShapeDtypeStruct((N_PAD, ACT), jnp.float32),
            jax.ShapeDtypeStruct((N_PAD, ACT), jnp.float32),
        ),
    )(o_pad, p_pad)


@functools.partial(
    pl.kernel,
    out_type=jax.ShapeDtypeStruct((N_PAD,), jnp.float32),
    mesh=_MESH,
    compiler_params=pltpu.CompilerParams(needs_layout_passes=False),
    scratch_types=[
        pltpu.VMEM((N_STATES,), jnp.float32),
        pltpu.VMEM((PER_W * ACT,), jnp.float32),
        pltpu.VMEM((PER_W * ACT,), jnp.float32),
        pltpu.VMEM((CHUNK_STATES * ACT * ACT,), jnp.int32),
        pltpu.VMEM((PER_W,), jnp.float32),
    ],
)
def _wr_step(wr_hbm, f_hbm, a_hbm, b_hbm, out_hbm, wr_v, a_v, b_v, f_v, out_v):
    wid = _worker_id()
    base = wid * PER_W
    pltpu.sync_copy(wr_hbm, wr_v)
    pltpu.sync_copy(a_hbm.at[pl.ds(base * ACT, PER_W * ACT)], a_v)
    pltpu.sync_copy(b_hbm.at[pl.ds(base * ACT, PER_W * ACT)], b_v)

    lanes = lax.iota(jnp.int32, 16)

    def chunk_body(ci, carry):
        pltpu.sync_copy(
            f_hbm.at[pl.ds((base + ci * CHUNK_STATES) * ACT * ACT,
                           CHUNK_STATES * ACT * ACT)],
            f_v)

        def group_body(gi, c2):
            soff = ci * CHUNK_STATES + gi * 16
            ab_base = (soff + lanes) * ACT
            b_cols = [plsc.load_gather(b_v, [ab_base + k]) for k in range(ACT)]
            f_base = (gi * 16 + lanes) * (ACT * ACT)
            acc = jnp.zeros((16,), jnp.float32)
            for j in range(ACT):
                a_j = plsc.load_gather(a_v, [ab_base + j])
                for k in range(ACT):
                    f_idx = plsc.load_gather(f_v, [f_base + (j * ACT + k)])
                    g = plsc.load_gather(wr_v, [f_idx])
                    acc = acc + (a_j * b_cols[k]) * g
            out_v[pl.ds(soff, 16)] = acc
            return c2

        lax.fori_loop(0, CHUNK_GROUPS, group_body, carry)
        return carry

    lax.fori_loop(0, N_CHUNKS, chunk_body, 0)
    pltpu.sync_copy(out_v, out_hbm.at[pl.ds(base, PER_W)])


def kernel(opponent_policy, policy, wr, F, iter_num):
    opp_t = opponent_policy.reshape(
        MAX_HEALTH + 1, MAX_HEALTH + 1, MAX_ENERGY + 1, MAX_ENERGY + 1, ACT
    ).transpose(1, 0, 3, 2, 4).reshape(N_STATES, ACT)

    pad = N_PAD - N_STATES
    o_pad = jnp.pad(opp_t, ((0, pad), (0, 0)))
    p_pad = jnp.pad(policy, ((0, pad), (0, 0)))
    f_pad = jnp.pad(F.reshape(N_STATES, ACT * ACT), ((0, pad), (0, 0))).reshape(-1)

    a_2d, b_2d = _softmax_prep(o_pad, p_pad)
    a_flat = a_2d.reshape(-1)
    b_flat = b_2d.reshape(-1)

    def body(_, cur_wr):
        out = _wr_step(cur_wr, f_pad, a_flat, b_flat)
        return out[:N_STATES]

    return lax.fori_loop(0, iter_num, body, wr)


# speedup vs baseline: 493.4512x; 1.1879x over previous
import functools

import jax
import jax.numpy as jnp
from jax import lax
from jax.experimental import pallas as pl
from jax.experimental.pallas import tpu as pltpu
from jax.experimental.pallas import tpu_sc as plsc

MAX_HEALTH = 13
MAX_ENERGY = 13
ACT = 16
N_STATES = (MAX_HEALTH + 1) ** 2 * (MAX_ENERGY + 1) ** 2

NUM_WORKERS = 32
PER_W = 1216
N_PAD = NUM_WORKERS * PER_W
GROUPS = PER_W // 16
CHUNK_STATES = 32
CHUNK_GROUPS = CHUNK_STATES // 16
N_CHUNKS = PER_W // CHUNK_STATES
CHUNK_WORDS = CHUNK_STATES * ACT * ACT

_MESH = plsc.VectorSubcoreMesh(core_axis_name="c", subcore_axis_name="s")


def _worker_id():
    return lax.axis_index("c") * 16 + lax.axis_index("s")


def _softmax_body(o_ref, p_ref, a_ref, b_ref):
    for src, dst in ((o_ref, a_ref), (p_ref, b_ref)):
        x = src[:]
        m = jnp.max(x, axis=1, keepdims=True)
        e = jnp.exp(x - m)
        dst[:] = e / jnp.sum(e, axis=1, keepdims=True)


def _softmax_prep(o_pad, p_pad):
    blk = pl.BlockSpec((512, ACT), lambda i: (i, 0))
    return pl.pallas_call(
        _softmax_body,
        grid=(N_PAD // 512,),
        in_specs=[blk, blk],
        out_specs=(blk, blk),
        out_shape=(
            jax.ShapeDtypeStruct((N_PAD, ACT), jnp.float32),
            jax.ShapeDtypeStruct((N_PAD, ACT), jnp.float32),
        ),
    )(o_pad, p_pad)


@functools.partial(
    pl.kernel,
    out_type=jax.ShapeDtypeStruct((N_PAD,), jnp.float32),
    mesh=_MESH,
    compiler_params=pltpu.CompilerParams(needs_layout_passes=False),
    scratch_types=[
        pltpu.VMEM((N_STATES,), jnp.float32),
        pltpu.VMEM((PER_W * ACT,), jnp.float32),
        pltpu.VMEM((PER_W * ACT,), jnp.float32),
        pltpu.VMEM((CHUNK_WORDS,), jnp.int32),
        pltpu.VMEM((CHUNK_WORDS,), jnp.int32),
        pltpu.VMEM((PER_W,), jnp.float32),
        pltpu.SemaphoreType.DMA,
        pltpu.SemaphoreType.DMA,
        pltpu.SemaphoreType.DMA,
    ],
)
def _wr_step(wr_hbm, f_hbm, a_hbm, b_hbm, out_hbm,
             wr_v, a_v, b_v, f0_v, f1_v, out_v, sem_s, sem0, sem1):
    wid = _worker_id()
    base = wid * PER_W
    f_base_w = base * ACT * ACT

    def f_chunk_src(ci):
        return f_hbm.at[pl.ds(f_base_w + ci * CHUNK_WORDS, CHUNK_WORDS)]

    pltpu.async_copy(f_chunk_src(0), f0_v, sem0)
    pltpu.async_copy(a_hbm.at[pl.ds(base * ACT, PER_W * ACT)], a_v, sem_s)
    pltpu.async_copy(b_hbm.at[pl.ds(base * ACT, PER_W * ACT)], b_v, sem_s)
    pltpu.async_copy(wr_hbm, wr_v, sem_s)
    pltpu.make_async_copy(a_hbm.at[pl.ds(0, PER_W * ACT)], a_v, sem_s).wait()
    pltpu.make_async_copy(b_hbm.at[pl.ds(0, PER_W * ACT)], b_v, sem_s).wait()
    pltpu.make_async_copy(wr_hbm, wr_v, sem_s).wait()

    lanes = lax.iota(jnp.int32, 16)
    f_lane_base = lanes * (ACT * ACT)

    def compute_chunk(ci, f_v):
        def group_body(gi, c2):
            soff = ci * CHUNK_STATES + gi * 16
            ab_base = (soff + lanes) * ACT
            b_cols = [plsc.load_gather(b_v, [ab_base + k]) for k in range(ACT)]
            f_base = f_lane_base + gi * (16 * ACT * ACT)

            def j_body(j, acc):
                a_j = plsc.load_gather(a_v, [ab_base + j])
                fb_j = f_base + j * ACT
                acc_e = jnp.zeros((16,), jnp.float32)
                acc_o = jnp.zeros((16,), jnp.float32)
                for k in range(0, ACT, 2):
                    f_e = plsc.load_gather(f_v, [fb_j + k])
                    f_o = plsc.load_gather(f_v, [fb_j + (k + 1)])
                    acc_e = acc_e + b_cols[k] * plsc.load_gather(wr_v, [f_e])
                    acc_o = acc_o + b_cols[k + 1] * plsc.load_gather(wr_v, [f_o])
                return acc + a_j * (acc_e + acc_o)

            acc = lax.fori_loop(0, ACT, j_body, jnp.zeros((16,), jnp.float32))
            out_v[pl.ds(soff, 16)] = acc
            return c2

        lax.fori_loop(0, CHUNK_GROUPS, group_body, 0)

    def pair_body(t, carry):
        c0 = t * 2
        pltpu.make_async_copy(f_chunk_src(0), f0_v, sem0).wait()
        pltpu.async_copy(f_chunk_src(c0 + 1), f1_v, sem1)
        compute_chunk(c0, f0_v)
        pltpu.make_async_copy(f_chunk_src(0), f1_v, sem1).wait()

        @pl.when(c0 + 2 < N_CHUNKS)
        def _():
            pltpu.async_copy(f_chunk_src(c0 + 2), f0_v, sem0)

        compute_chunk(c0 + 1, f1_v)
        return carry

    lax.fori_loop(0, N_CHUNKS // 2, pair_body, 0)
    pltpu.sync_copy(out_v, out_hbm.at[pl.ds(base, PER_W)])


def kernel(opponent_policy, policy, wr, F, iter_num):
    opp_t = opponent_policy.reshape(
        MAX_HEALTH + 1, MAX_HEALTH + 1, MAX_ENERGY + 1, MAX_ENERGY + 1, ACT
    ).transpose(1, 0, 3, 2, 4).reshape(N_STATES, ACT)

    pad = N_PAD - N_STATES
    o_pad = jnp.pad(opp_t, ((0, pad), (0, 0)))
    p_pad = jnp.pad(policy, ((0, pad), (0, 0)))
    f_pad = jnp.pad(F.reshape(N_STATES, ACT * ACT), ((0, pad), (0, 0))).reshape(-1)

    a_2d, b_2d = _softmax_prep(o_pad, p_pad)
    a_flat = a_2d.reshape(-1)
    b_flat = b_2d.reshape(-1)

    def body(_, cur_wr):
        out = _wr_step(cur_wr, f_pad, a_flat, b_flat)
        return out[:N_STATES]

    return lax.fori_loop(0, iter_num, body, wr)


# speedup vs baseline: 682.0420x; 1.3822x over previous
import functools

import jax
import jax.numpy as jnp
from jax import lax
from jax.experimental import pallas as pl
from jax.experimental.pallas import tpu as pltpu
from jax.experimental.pallas import tpu_sc as plsc

MAX_HEALTH = 13
MAX_ENERGY = 13
ACT = 16
N_STATES = (MAX_HEALTH + 1) ** 2 * (MAX_ENERGY + 1) ** 2

NUM_WORKERS = 32
PER_W = 1216
N_PAD = NUM_WORKERS * PER_W
GROUPS = PER_W // 16
CHUNK_STATES = 32
CHUNK_GROUPS = CHUNK_STATES // 16
N_CHUNKS = PER_W // CHUNK_STATES
CHUNK_WORDS = CHUNK_STATES * ACT * ACT // 2
GROUP_WORDS = 16 * ACT * ACT // 2

_MESH = plsc.VectorSubcoreMesh(core_axis_name="c", subcore_axis_name="s")


def _worker_id():
    return lax.axis_index("c") * 16 + lax.axis_index("s")


def _softmax_body(o_ref, p_ref, a_ref, b_ref):
    for src, dst in ((o_ref, a_ref), (p_ref, b_ref)):
        x = src[:]
        m = jnp.max(x, axis=1, keepdims=True)
        e = jnp.exp(x - m)
        dst[:] = e / jnp.sum(e, axis=1, keepdims=True)


def _softmax_prep(o_pad, p_pad):
    blk = pl.BlockSpec((512, ACT), lambda i: (i, 0))
    return pl.pallas_call(
        _softmax_body,
        grid=(N_PAD // 512,),
        in_specs=[blk, blk],
        out_specs=(blk, blk),
        out_shape=(
            jax.ShapeDtypeStruct((N_PAD, ACT), jnp.float32),
            jax.ShapeDtypeStruct((N_PAD, ACT), jnp.float32),
        ),
    )(o_pad, p_pad)


@functools.partial(
    pl.kernel,
    out_type=jax.ShapeDtypeStruct((N_PAD,), jnp.float32),
    mesh=_MESH,
    compiler_params=pltpu.CompilerParams(needs_layout_passes=False),
    scratch_types=[
        pltpu.VMEM((N_STATES,), jnp.float32),
        pltpu.VMEM((PER_W * ACT,), jnp.float32),
        pltpu.VMEM((PER_W * ACT,), jnp.float32),
        pltpu.VMEM((CHUNK_WORDS,), jnp.int32),
        pltpu.VMEM((CHUNK_WORDS,), jnp.int32),
        pltpu.VMEM((PER_W,), jnp.float32),
        pltpu.SemaphoreType.DMA,
        pltpu.SemaphoreType.DMA,
        pltpu.SemaphoreType.DMA,
    ],
)
def _wr_step(wr_hbm, f_hbm, a_hbm, b_hbm, out_hbm,
             wr_v, a_v, b_v, f0_v, f1_v, out_v, sem_s, sem0, sem1):
    wid = _worker_id()
    base = wid * PER_W
    f_base_w = base * ACT * ACT // 2

    def f_chunk_src(ci):
        off = pl.multiple_of(f_base_w + ci * CHUNK_WORDS, 128)
        return f_hbm.at[pl.ds(off, CHUNK_WORDS)]

    pltpu.async_copy(f_chunk_src(0), f0_v, sem0)
    for r in range(ACT):
        pltpu.async_copy(a_hbm.at[pl.ds(r * N_PAD + base, PER_W)],
                         a_v.at[pl.ds(r * PER_W, PER_W)], sem_s)
        pltpu.async_copy(b_hbm.at[pl.ds(r * N_PAD + base, PER_W)],
                         b_v.at[pl.ds(r * PER_W, PER_W)], sem_s)
    pltpu.async_copy(wr_hbm, wr_v, sem_s)
    for r in range(ACT):
        pltpu.make_async_copy(a_hbm.at[pl.ds(0, PER_W)],
                              a_v.at[pl.ds(0, PER_W)], sem_s).wait()
        pltpu.make_async_copy(b_hbm.at[pl.ds(0, PER_W)],
                              b_v.at[pl.ds(0, PER_W)], sem_s).wait()
    pltpu.make_async_copy(wr_hbm, wr_v, sem_s).wait()

    def compute_chunk(ci, f_v):
        def group_body(gi, c2):
            soff = ci * CHUNK_STATES + gi * 16
            b_cols = [b_v[pl.ds(k * PER_W + soff, 16)] for k in range(ACT)]
            f_base = gi * GROUP_WORDS

            def j_body(j, acc):
                a_j = a_v[pl.ds(pl.multiple_of(j * PER_W + soff, 16), 16)]
                jb = pl.multiple_of(f_base + j * (ACT // 2) * 16, 16)
                acc_e = jnp.zeros((16,), jnp.float32)
                acc_o = jnp.zeros((16,), jnp.float32)
                for p in range(ACT // 2):
                    x = f_v[pl.ds(jb + p * 16, 16)]
                    lo = x & jnp.int32(0xFFFF)
                    hi = lax.shift_right_logical(x, 16)
                    acc_e = acc_e + b_cols[2 * p] * plsc.load_gather(wr_v, [lo])
                    acc_o = acc_o + b_cols[2 * p + 1] * plsc.load_gather(wr_v, [hi])
                return acc + a_j * (acc_e + acc_o)

            acc = lax.fori_loop(0, ACT, j_body, jnp.zeros((16,), jnp.float32))
            out_v[pl.ds(soff, 16)] = acc
            return c2

        lax.fori_loop(0, CHUNK_GROUPS, group_body, 0)

    def pair_body(t, carry):
        c0 = t * 2
        pltpu.make_async_copy(f_chunk_src(0), f0_v, sem0).wait()
        pltpu.async_copy(f_chunk_src(c0 + 1), f1_v, sem1)
        compute_chunk(c0, f0_v)
        pltpu.make_async_copy(f_chunk_src(0), f1_v, sem1).wait()

        @pl.when(c0 + 2 < N_CHUNKS)
        def _():
            pltpu.async_copy(f_chunk_src(c0 + 2), f0_v, sem0)

        compute_chunk(c0 + 1, f1_v)
        return carry

    lax.fori_loop(0, N_CHUNKS // 2, pair_body, 0)
    pltpu.sync_copy(out_v, out_hbm.at[pl.ds(base, PER_W)])


def kernel(opponent_policy, policy, wr, F, iter_num):
    opp_t = opponent_policy.reshape(
        MAX_HEALTH + 1, MAX_HEALTH + 1, MAX_ENERGY + 1, MAX_ENERGY + 1, ACT
    ).transpose(1, 0, 3, 2, 4).reshape(N_STATES, ACT)

    pad = N_PAD - N_STATES
    o_pad = jnp.pad(opp_t, ((0, pad), (0, 0)))
    p_pad = jnp.pad(policy, ((0, pad), (0, 0)))
    fr = jnp.pad(F.reshape(N_STATES, ACT * ACT), ((0, pad), (0, 0)))
    fr = fr.reshape(N_PAD // 16, 16, ACT * ACT // 2, 2)
    packed = fr[..., 0] | (fr[..., 1] << 16)
    f_pad = packed.transpose(0, 2, 1).reshape(-1)

    a_2d, b_2d = _softmax_prep(o_pad, p_pad)
    a_flat = a_2d.T.reshape(-1)
    b_flat = b_2d.T.reshape(-1)

    def body(_, cur_wr):
        out = _wr_step(cur_wr, f_pad, a_flat, b_flat)
        return out[:N_STATES]

    return lax.fori_loop(0, iter_num, body, wr)


# speedup vs baseline: 898.7715x; 1.3178x over previous
import functools

import jax
import jax.numpy as jnp
from jax import lax
from jax.experimental import pallas as pl
from jax.experimental.pallas import tpu as pltpu
from jax.experimental.pallas import tpu_sc as plsc

MAX_HEALTH = 13
MAX_ENERGY = 13
ACT = 16
N_STATES = (MAX_HEALTH + 1) ** 2 * (MAX_ENERGY + 1) ** 2

NUM_WORKERS = 32
PER_W = 1216
N_PAD = NUM_WORKERS * PER_W
GROUPS = PER_W // 16
CHUNK_STATES = 32
CHUNK_GROUPS = CHUNK_STATES // 16
N_CHUNKS = PER_W // CHUNK_STATES
CHUNK_WORDS = CHUNK_STATES * ACT * ACT // 2
GROUP_WORDS = 16 * ACT * ACT // 2

_MESH = plsc.VectorSubcoreMesh(core_axis_name="c", subcore_axis_name="s")


def _worker_id():
    return lax.axis_index("c") * 16 + lax.axis_index("s")


_PREP_ROWS = 256


def _prep_body(o_ref, p_ref, f_ref, a_ref, b_ref, fp_ref):
    for src, dst in ((o_ref, a_ref), (p_ref, b_ref)):
        x = src[:]
        m = jnp.max(x, axis=1, keepdims=True)
        e = jnp.exp(x - m)
        d = e / jnp.sum(e, axis=1, keepdims=True)
        dst[:] = d.T
    f = f_ref[:]
    for g in range(_PREP_ROWS // 16):
        y = f[g * 16:(g + 1) * 16, :].T
        fp_ref[g * 128:(g + 1) * 128, :] = y[0:128, :] | (y[128:256, :] << 16)


def _prep(o_pad, p_pad, f_pad2d):
    io_blk = pl.BlockSpec((_PREP_ROWS, ACT), lambda i: (i, 0))
    return pl.pallas_call(
        _prep_body,
        grid=(N_PAD // _PREP_ROWS,),
        in_specs=[io_blk, io_blk,
                  pl.BlockSpec((_PREP_ROWS, ACT * ACT), lambda i: (i, 0))],
        out_specs=(pl.BlockSpec((ACT, _PREP_ROWS), lambda i: (0, i)),
                   pl.BlockSpec((ACT, _PREP_ROWS), lambda i: (0, i)),
                   pl.BlockSpec((_PREP_ROWS * 8, ACT), lambda i: (i, 0))),
        out_shape=(
            jax.ShapeDtypeStruct((ACT, N_PAD), jnp.float32),
            jax.ShapeDtypeStruct((ACT, N_PAD), jnp.float32),
            jax.ShapeDtypeStruct((N_PAD * 8, ACT), jnp.int32),
        ),
    )(o_pad, p_pad, f_pad2d)


@functools.partial(
    pl.kernel,
    out_type=jax.ShapeDtypeStruct((N_PAD,), jnp.float32),
    mesh=_MESH,
    compiler_params=pltpu.CompilerParams(needs_layout_passes=False),
    scratch_types=[
        pltpu.VMEM((N_STATES,), jnp.float32),
        pltpu.VMEM((PER_W * ACT,), jnp.float32),
        pltpu.VMEM((PER_W * ACT,), jnp.float32),
        pltpu.VMEM((CHUNK_WORDS,), jnp.int32),
        pltpu.VMEM((CHUNK_WORDS,), jnp.int32),
        pltpu.VMEM((PER_W,), jnp.float32),
        pltpu.SemaphoreType.DMA,
        pltpu.SemaphoreType.DMA,
        pltpu.SemaphoreType.DMA,
    ],
)
def _wr_step(wr_hbm, f_hbm, a_hbm, b_hbm, out_hbm,
             wr_v, a_v, b_v, f0_v, f1_v, out_v, sem_s, sem0, sem1):
    wid = _worker_id()
    base = wid * PER_W
    f_base_w = base * ACT * ACT // 2

    def f_chunk_src(ci):
        off = pl.multiple_of(f_base_w + ci * CHUNK_WORDS, 128)
        return f_hbm.at[pl.ds(off, CHUNK_WORDS)]

    pltpu.async_copy(f_chunk_src(0), f0_v, sem0)
    for r in range(ACT):
        pltpu.async_copy(a_hbm.at[pl.ds(r * N_PAD + base, PER_W)],
                         a_v.at[pl.ds(r * PER_W, PER_W)], sem_s)
        pltpu.async_copy(b_hbm.at[pl.ds(r * N_PAD + base, PER_W)],
                         b_v.at[pl.ds(r * PER_W, PER_W)], sem_s)
    pltpu.async_copy(wr_hbm.at[pl.ds(0, N_STATES)], wr_v, sem_s)
    for r in range(ACT):
        pltpu.make_async_copy(a_hbm.at[pl.ds(0, PER_W)],
                              a_v.at[pl.ds(0, PER_W)], sem_s).wait()
        pltpu.make_async_copy(b_hbm.at[pl.ds(0, PER_W)],
                              b_v.at[pl.ds(0, PER_W)], sem_s).wait()
    pltpu.make_async_copy(wr_hbm.at[pl.ds(0, N_STATES)], wr_v, sem_s).wait()

    def compute_chunk(ci, f_v):
        def group_body(gi, c2):
            soff = ci * CHUNK_STATES + gi * 16
            b_cols = [b_v[pl.ds(k * PER_W + soff, 16)] for k in range(ACT)]
            f_base = gi * GROUP_WORDS

            def one_j(j):
                a_lo = a_v[pl.ds(pl.multiple_of(j * PER_W + soff, 16), 16)]
                a_hi = a_v[pl.ds(pl.multiple_of((j + 8) * PER_W + soff, 16), 16)]
                jb = pl.multiple_of(f_base + j * ACT * 16, 16)
                acc_e = jnp.zeros((16,), jnp.float32)
                acc_o = jnp.zeros((16,), jnp.float32)
                for k in range(ACT):
                    x = f_v[pl.ds(jb + k * 16, 16)]
                    lo = x & jnp.int32(0xFFFF)
                    hi = lax.shift_right_logical(x, 16)
                    acc_e = acc_e + b_cols[k] * plsc.load_gather(wr_v, [lo])
                    acc_o = acc_o + b_cols[k] * plsc.load_gather(wr_v, [hi])
                return a_lo * acc_e + a_hi * acc_o

            def j_body(j2, acc):
                return acc + one_j(2 * j2) + one_j(2 * j2 + 1)

            acc = lax.fori_loop(0, ACT // 4, j_body, jnp.zeros((16,), jnp.float32))
            out_v[pl.ds(soff, 16)] = acc
            return c2

        lax.fori_loop(0, CHUNK_GROUPS, group_body, 0)

    def pair_body(t, carry):
        c0 = t * 2
        pltpu.make_async_copy(f_chunk_src(0), f0_v, sem0).wait()
        pltpu.async_copy(f_chunk_src(c0 + 1), f1_v, sem1)
        compute_chunk(c0, f0_v)
        pltpu.make_async_copy(f_chunk_src(0), f1_v, sem1).wait()

        @pl.when(c0 + 2 < N_CHUNKS)
        def _():
            pltpu.async_copy(f_chunk_src(c0 + 2), f0_v, sem0)

        compute_chunk(c0 + 1, f1_v)
        return carry

    lax.fori_loop(0, N_CHUNKS // 2, pair_body, 0)
    pltpu.sync_copy(out_v, out_hbm.at[pl.ds(base, PER_W)])


def kernel(opponent_policy, policy, wr, F, iter_num):
    opp_t = opponent_policy.reshape(
        MAX_HEALTH + 1, MAX_HEALTH + 1, MAX_ENERGY + 1, MAX_ENERGY + 1, ACT
    ).transpose(1, 0, 3, 2, 4).reshape(N_STATES, ACT)

    pad = N_PAD - N_STATES
    o_pad = jnp.pad(opp_t, ((0, pad), (0, 0)))
    p_pad = jnp.pad(policy, ((0, pad), (0, 0)))
    f_pad2d = jnp.pad(F.reshape(N_STATES, ACT * ACT), ((0, pad), (0, 0)))

    a_2d, b_2d, f_packed = _prep(o_pad, p_pad, f_pad2d)
    a_flat = a_2d.reshape(-1)
    b_flat = b_2d.reshape(-1)
    f_pad = f_packed.reshape(-1)

    def body(_, cur_wr):
        return _wr_step(cur_wr, f_pad, a_flat, b_flat)

    wr0 = jnp.pad(wr, (0, pad))
    return lax.fori_loop(0, iter_num, body, wr0)[:N_STATES]


# speedup vs baseline: 908.0918x; 1.0104x over previous
import functools

import jax
import jax.numpy as jnp
from jax import lax
from jax.experimental import pallas as pl
from jax.experimental.pallas import tpu as pltpu
from jax.experimental.pallas import tpu_sc as plsc

MAX_HEALTH = 13
MAX_ENERGY = 13
ACT = 16
N_STATES = (MAX_HEALTH + 1) ** 2 * (MAX_ENERGY + 1) ** 2

NUM_WORKERS = 32
PER_W = 1216
N_PAD = NUM_WORKERS * PER_W
GROUPS = PER_W // 16
CHUNK_STATES = 32
CHUNK_GROUPS = CHUNK_STATES // 16
N_CHUNKS = PER_W // CHUNK_STATES
CHUNK_WORDS = CHUNK_STATES * ACT * ACT // 2
GROUP_WORDS = 16 * ACT * ACT // 2

_MESH = plsc.VectorSubcoreMesh(core_axis_name="c", subcore_axis_name="s")


def _worker_id():
    return lax.axis_index("c") * 16 + lax.axis_index("s")


_PREP_ROWS = 256


def _prep_body(o_ref, p_ref, f_ref, a_ref, b_ref, fp_ref):
    for src, dst in ((o_ref, a_ref), (p_ref, b_ref)):
        x = src[:]
        m = jnp.max(x, axis=1, keepdims=True)
        e = jnp.exp(x - m)
        d = e / jnp.sum(e, axis=1, keepdims=True)
        dst[:] = d.T
    f = f_ref[:]
    packed = f[:, 0:128] | (f[:, 128:256] << 16)
    pf = lax.bitcast_convert_type(packed, jnp.float32)
    for g in range(_PREP_ROWS // 16):
        y = pf[g * 16:(g + 1) * 16, :].T
        fp_ref[g * 128:(g + 1) * 128, :] = lax.bitcast_convert_type(y, jnp.int32)


def _prep(o_pad, p_pad, f_pad2d):
    io_blk = pl.BlockSpec((_PREP_ROWS, ACT), lambda i: (i, 0))
    return pl.pallas_call(
        _prep_body,
        grid=(N_PAD // _PREP_ROWS,),
        in_specs=[io_blk, io_blk,
                  pl.BlockSpec((_PREP_ROWS, ACT * ACT), lambda i: (i, 0))],
        out_specs=(pl.BlockSpec((ACT, _PREP_ROWS), lambda i: (0, i)),
                   pl.BlockSpec((ACT, _PREP_ROWS), lambda i: (0, i)),
                   pl.BlockSpec((_PREP_ROWS * 8, ACT), lambda i: (i, 0))),
        out_shape=(
            jax.ShapeDtypeStruct((ACT, N_PAD), jnp.float32),
            jax.ShapeDtypeStruct((ACT, N_PAD), jnp.float32),
            jax.ShapeDtypeStruct((N_PAD * 8, ACT), jnp.int32),
        ),
    )(o_pad, p_pad, f_pad2d)


@functools.partial(
    pl.kernel,
    out_type=jax.ShapeDtypeStruct((N_PAD,), jnp.float32),
    mesh=_MESH,
    compiler_params=pltpu.CompilerParams(needs_layout_passes=False),
    scratch_types=[
        pltpu.VMEM((N_STATES,), jnp.float32),
        pltpu.VMEM((PER_W * ACT,), jnp.float32),
        pltpu.VMEM((PER_W * ACT,), jnp.float32),
        pltpu.VMEM((CHUNK_WORDS,), jnp.int32),
        pltpu.VMEM((CHUNK_WORDS,), jnp.int32),
        pltpu.VMEM((CHUNK_WORDS,), jnp.int32),
        pltpu.VMEM((CHUNK_WORDS,), jnp.int32),
        pltpu.VMEM((PER_W,), jnp.float32),
        pltpu.SemaphoreType.DMA,
        pltpu.SemaphoreType.DMA,
        pltpu.SemaphoreType.DMA,
        pltpu.SemaphoreType.DMA,
        pltpu.SemaphoreType.DMA,
    ],
)
def _wr_step(wr_hbm, f_hbm, a_hbm, b_hbm, out_hbm,
             wr_v, a_v, b_v, f0_v, f1_v, f2_v, f3_v, out_v,
             sem_s, sem0, sem1, sem2, sem3):
    wid = _worker_id()
    base = wid * PER_W
    f_base_w = base * ACT * ACT // 2

    def f_chunk_src(ci):
        off = pl.multiple_of(f_base_w + ci * CHUNK_WORDS, 128)
        return f_hbm.at[pl.ds(off, CHUNK_WORDS)]

    ring = ((f0_v, sem0), (f1_v, sem1), (f2_v, sem2), (f3_v, sem3))

    for c in range(3):
        pltpu.async_copy(f_chunk_src(c), ring[c][0], ring[c][1])
    for r in range(ACT):
        pltpu.async_copy(a_hbm.at[pl.ds(r * N_PAD + base, PER_W)],
                         a_v.at[pl.ds(r * PER_W, PER_W)], sem_s)
        pltpu.async_copy(b_hbm.at[pl.ds(r * N_PAD + base, PER_W)],
                         b_v.at[pl.ds(r * PER_W, PER_W)], sem_s)
    pltpu.async_copy(wr_hbm.at[pl.ds(0, N_STATES)], wr_v, sem_s)
    for r in range(ACT):
        pltpu.make_async_copy(a_hbm.at[pl.ds(0, PER_W)],
                              a_v.at[pl.ds(0, PER_W)], sem_s).wait()
        pltpu.make_async_copy(b_hbm.at[pl.ds(0, PER_W)],
                              b_v.at[pl.ds(0, PER_W)], sem_s).wait()
    pltpu.make_async_copy(wr_hbm.at[pl.ds(0, N_STATES)], wr_v, sem_s).wait()

    def compute_chunk(ci, f_v):
        def group_body(gi, c2):
            soff = ci * CHUNK_STATES + gi * 16
            b_cols = [b_v[pl.ds(k * PER_W + soff, 16)] for k in range(ACT)]
            f_base = gi * GROUP_WORDS

            def one_j(j):
                a_lo = a_v[pl.ds(pl.multiple_of(j * PER_W + soff, 16), 16)]
                a_hi = a_v[pl.ds(pl.multiple_of((j + 8) * PER_W + soff, 16), 16)]
                jb = pl.multiple_of(f_base + j * ACT * 16, 16)
                acc_e = jnp.zeros((16,), jnp.float32)
                acc_o = jnp.zeros((16,), jnp.float32)
                for k in range(ACT):
                    x = f_v[pl.ds(jb + k * 16, 16)]
                    lo = x & jnp.int32(0xFFFF)
                    hi = lax.shift_right_logical(x, 16)
                    acc_e = acc_e + b_cols[k] * plsc.load_gather(wr_v, [lo])
                    acc_o = acc_o + b_cols[k] * plsc.load_gather(wr_v, [hi])
                return a_lo * acc_e + a_hi * acc_o

            def j_body(j2, acc):
                return acc + one_j(2 * j2) + one_j(2 * j2 + 1)

            acc = lax.fori_loop(0, ACT // 4, j_body, jnp.zeros((16,), jnp.float32))
            out_v[pl.ds(soff, 16)] = acc
            return c2

        lax.fori_loop(0, CHUNK_GROUPS, group_body, 0)

    def quad_body(t, carry):
        c0 = t * 4
        for b in range(4):
            c = c0 + b
            f_v, sem_b = ring[b]
            pltpu.make_async_copy(f_chunk_src(0), f_v, sem_b).wait()

            @pl.when(c + 3 < N_CHUNKS)
            def _():
                nb = (b + 3) % 4
                pltpu.async_copy(f_chunk_src(c + 3), ring[nb][0], ring[nb][1])

            compute_chunk(c, f_v)
        return carry

    lax.fori_loop(0, N_CHUNKS // 4, quad_body, 0)
    for b in range(N_CHUNKS % 4):
        c = (N_CHUNKS // 4) * 4 + b
        f_v, sem_b = ring[b]
        pltpu.make_async_copy(f_chunk_src(0), f_v, sem_b).wait()
        compute_chunk(c, f_v)
    pltpu.sync_copy(out_v, out_hbm.at[pl.ds(base, PER_W)])


def kernel(opponent_policy, policy, wr, F, iter_num):
    opp_t = opponent_policy.reshape(
        MAX_HEALTH + 1, MAX_HEALTH + 1, MAX_ENERGY + 1, MAX_ENERGY + 1, ACT
    ).transpose(1, 0, 3, 2, 4).reshape(N_STATES, ACT)

    pad = N_PAD - N_STATES
    o_pad = jnp.pad(opp_t, ((0, pad), (0, 0)))
    p_pad = jnp.pad(policy, ((0, pad), (0, 0)))
    f_pad2d = jnp.pad(F.reshape(N_STATES, ACT * ACT), ((0, pad), (0, 0)))

    a_2d, b_2d, f_packed = _prep(o_pad, p_pad, f_pad2d)
    a_flat = a_2d.reshape(-1)
    b_flat = b_2d.reshape(-1)
    f_pad = f_packed.reshape(-1)

    def body(_, cur_wr):
        return _wr_step(cur_wr, f_pad, a_flat, b_flat)

    wr0 = jnp.pad(wr, (0, pad))
    return lax.fori_loop(0, iter_num, body, wr0)[:N_STATES]
